# Initial kernel scaffold; baseline (speedup 1.0000x reference)
import jax
import jax.numpy as jnp
from jax.experimental import pallas as pl


def kernel(t_in, y_in, inds_1r, inds_1p, inds_2r, inds_2p, A1, E1, A2, E2):
    raise NotImplementedError("write your pallas kernel here")


# speedup vs baseline: 1.7793x; 1.7793x over previous
import functools

import jax
import jax.numpy as jnp
from jax import lax
from jax.experimental import pallas as pl
from jax.experimental.pallas import tpu as pltpu
from jax.experimental.pallas import tpu_sc as plsc

N_SPEC = 1024
B = 1024
R1_N = 4096
R2_N = 16384
RTOT = R1_N + R2_N
RBLK = 4096
NBLK = RTOT // RBLK
NC = 2
NS = 16
NW = NC * NS
BPW = B // NW
L = 16
YW = (N_SPEC + 1) * BPW
AW = N_SPEC * BPW


def _build_sc_kernel():
    mesh = plsc.VectorSubcoreMesh(core_axis_name="c", subcore_axis_name="s")

    @functools.partial(
        pl.kernel,
        mesh=mesh,
        out_type=jax.ShapeDtypeStruct((NW, AW), jnp.float32),
        scratch_types=[
            pltpu.VMEM((YW,), jnp.float32),
            pltpu.VMEM((AW,), jnp.float32),
            pltpu.VMEM((RBLK,), jnp.int32),
            pltpu.VMEM((RBLK,), jnp.int32),
            pltpu.VMEM((RBLK,), jnp.int32),
            pltpu.VMEM((RBLK,), jnp.float32),
            pltpu.VMEM((RBLK,), jnp.float32),
            pltpu.VMEM((BPW,), jnp.float32),
        ],
    )
    def reaction_kernel(yr_hbm, t_hbm, i_hbm, j_hbm, p_hbm, a_hbm, e_hbm,
                        out_hbm, y_v, acc_v, i_v, j_v, p_v, a_v, e_v, t_v):
        wid = lax.axis_index("s") * NC + lax.axis_index("c")

        pltpu.sync_copy(yr_hbm.at[wid], y_v)
        pltpu.sync_copy(t_hbm.at[pl.ds(wid * BPW, BPW)], t_v)

        def zero_body(s, carry):
            acc_v[pl.ds(s * L, L)] = jnp.zeros((L,), jnp.float32)
            return carry
        lax.fori_loop(0, AW // L, zero_body, 0)

        invt0 = 1.0 / t_v[pl.ds(0, L)]
        invt1 = 1.0 / t_v[pl.ds(L, L)]

        for blk in range(NBLK):
            base = blk * RBLK
            pltpu.sync_copy(i_hbm.at[pl.ds(base, RBLK)], i_v)
            pltpu.sync_copy(j_hbm.at[pl.ds(base, RBLK)], j_v)
            pltpu.sync_copy(p_hbm.at[pl.ds(base, RBLK)], p_v)
            pltpu.sync_copy(a_hbm.at[pl.ds(base, RBLK)], a_v)
            pltpu.sync_copy(e_hbm.at[pl.ds(base, RBLK)], e_v)

            def body(g, carry):
                it0, it1 = carry
                gb = g * L
                iv16 = i_v[pl.ds(gb, L)] * BPW
                jv16 = j_v[pl.ds(gb, L)] * BPW
                pv16 = p_v[pl.ds(gb, L)] * BPW
                av16 = a_v[pl.ds(gb, L)]
                ev16 = e_v[pl.ds(gb, L)]
                for k in range(L):
                    i = iv16[k]
                    j = jv16[k]
                    p = pv16[k]
                    a = av16[k]
                    e = ev16[k]
                    yi0 = y_v[pl.ds(i, L)]
                    yj0 = y_v[pl.ds(j, L)]
                    term0 = yi0 * yj0 * (a * jnp.exp(-e * it0))
                    plsc.addupdate(acc_v.at[pl.ds(p, L)], term0)
                    yi1 = y_v[pl.ds(i + L, L)]
                    yj1 = y_v[pl.ds(j + L, L)]
                    term1 = yi1 * yj1 * (a * jnp.exp(-e * it1))
                    plsc.addupdate(acc_v.at[pl.ds(p + L, L)], term1)
                return carry
            lax.fori_loop(0, RBLK // L, body, (invt0, invt1))

        pltpu.sync_copy(acc_v, out_hbm.at[wid])

    return reaction_kernel


_SC_KERNEL = _build_sc_kernel()


def kernel(t_in, y_in, inds_1r, inds_1p, inds_2r, inds_2p, A1, E1, A2, E2):
    iv = jnp.concatenate([inds_1r, inds_2r[:, 0]])
    jv = jnp.concatenate([jnp.full((R1_N,), N_SPEC, jnp.int32), inds_2r[:, 1]])
    pv = jnp.concatenate([inds_1p, inds_2p])
    av = jnp.concatenate([A1, A2])
    ev = jnp.concatenate([E1, E2])
    y_aug = jnp.concatenate(
        [y_in, jnp.ones((B, 1), jnp.float32)], axis=1)
    yr = y_aug.reshape(NW, BPW, N_SPEC + 1).transpose(0, 2, 1).reshape(NW, YW)
    tflat = t_in.reshape(B)

    out = _SC_KERNEL(yr, tflat, iv, jv, pv, av, ev)
    return out.reshape(NW, N_SPEC, BPW).transpose(0, 2, 1).reshape(B, N_SPEC)
